# baseline (speedup 1.0000x reference)
import jax
import jax.numpy as jnp
from jax import lax
from jax.experimental import pallas as pl
from jax.experimental.pallas import tpu as pltpu
from jax.experimental.pallas import tpu_sc as plsc

VOCAB = 100000
D = 64
BATCH = 4096
SEQ = 200
NC, NS = 2, 16
NW = NC * NS
BW = BATCH // NW
SCALE = 8.0


def _sc_body(idxT_hbm, pos_hbm, table_hbm, out_hbm,
             idx_v, buf_g, buf_o, pos_v,
             sem_i0, sem_i1, sem_i2, sem_i3,
             sem_g0, sem_g1, sem_g2, sem_g3, sem_o0, sem_o1):
    wid = lax.axis_index("s") * NC + lax.axis_index("c")
    b0 = wid * BW
    sem_i = (sem_i0, sem_i1, sem_i2, sem_i3)
    sem_g = (sem_g0, sem_g1, sem_g2, sem_g3)
    sem_o = (sem_o0, sem_o1)

    pltpu.sync_copy(pos_hbm, pos_v)

    lanes = lax.iota(jnp.int32, 16)
    bvecs = [lanes + 16 * q for q in range(BW // 16)]

    def idx_fetch(s, u):
        return pltpu.make_async_copy(
            idxT_hbm.at[s, pl.ds(b0, BW)], idx_v.at[u], sem_i[u])

    def gather(u):
        return pltpu.make_async_copy(
            table_hbm.at[idx_v.at[u]], buf_g.at[u], sem_g[u])

    def writebacks(s, u):
        return [
            pltpu.make_async_copy(
                buf_o.at[u], out_hbm.at[s, :, wid], sem_o[u])
        ]

    def compute(s, u, uo):
        s_vec = jnp.zeros((16,), jnp.int32) + s

        @plsc.parallel_loop(0, 4 * 16, unroll=2)
        def _j(j):
            cvec = ((lanes + j) & 15) + (j & 48)
            pvr = plsc.load_gather(pos_v, [s_vec, cvec])
            for q in range(BW // 16):
                x = plsc.load_gather(buf_g.at[u], [bvecs[q], cvec])
                plsc.store_scatter(
                    buf_o.at[uo], [cvec >> 3, cvec & 7, bvecs[q]],
                    x * SCALE + pvr)

    for u in range(4):
        idx_fetch(u, u).start()
    for u in range(4):
        idx_fetch(u, u).wait()
        gather(u).start()

    @pl.loop(0, SEQ, step=4)
    def _s(g):
        for u in range(4):
            s = g + u
            uo = u % 2
            gather(u).wait()

            @pl.when(s + 4 < SEQ)
            def _prefetch_idx():
                idx_fetch(s + 4, u).start()

            @pl.when(s >= 2)
            def _wait_prev_writeback():
                for wb in writebacks(s, uo):
                    wb.wait()

            compute(s, u, uo)
            for wb in writebacks(s, uo):
                wb.start()

            @pl.when(s + 4 < SEQ)
            def _next_gather():
                idx_fetch(s + 4, u).wait()
                gather(u).start()

    for u in range(2):
        for wb in writebacks(SEQ - 2 + u, u):
            wb.wait()


@jax.jit
def _embed(idxT, pos200, table):
    mesh = plsc.VectorSubcoreMesh(core_axis_name="c", subcore_axis_name="s")
    kfn = pl.kernel(
        _sc_body,
        out_type=jax.ShapeDtypeStruct((SEQ, 8, NW, 8, BW), jnp.float32),
        mesh=mesh,
        scratch_types=[
            pltpu.VMEM((4, BW), jnp.int32),
            pltpu.VMEM((4, BW, D), jnp.float32),
            pltpu.VMEM((2, 8, 8, BW), jnp.float32),
            pltpu.VMEM((SEQ, D), jnp.float32),
        ] + [pltpu.SemaphoreType.DMA] * 10,
        compiler_params=pltpu.CompilerParams(
            needs_layout_passes=False,
            use_tc_tiling_on_sc=False,
        ),
    )
    return kfn(idxT, pos200, table)


def kernel(inputs, table, pos_encoding):
    idxT = inputs.T
    pos200 = pos_encoding[:SEQ]
    out5 = _embed(idxT, pos200, table)
    return out5.transpose(2, 4, 0, 1, 3).reshape(BATCH, SEQ, D)


# speedup vs baseline: 1.0042x; 1.0042x over previous
import jax
import jax.numpy as jnp
from jax import lax
from jax.experimental import pallas as pl
from jax.experimental.pallas import tpu as pltpu
from jax.experimental.pallas import tpu_sc as plsc

VOCAB = 100000
D = 64
BATCH = 4096
SEQ = 200
NC, NS = 2, 16
NW = NC * NS
BW = BATCH // NW
SCALE = 8.0


def _sc_body(idxT_hbm, pos_hbm, table_hbm, out_hbm,
             idx_v, buf_g, buf_o, pos_v,
             sem_i0, sem_i1, sem_i2, sem_i3,
             sem_g0, sem_g1, sem_g2, sem_g3, sem_o0, sem_o1):
    wid = lax.axis_index("s") * NC + lax.axis_index("c")
    b0 = wid * BW
    sem_i = (sem_i0, sem_i1, sem_i2, sem_i3)
    sem_g = (sem_g0, sem_g1, sem_g2, sem_g3)
    sem_o = (sem_o0, sem_o1)

    pltpu.sync_copy(pos_hbm, pos_v)

    lanes = lax.iota(jnp.int32, 16)
    bvecs = [lanes + 16 * q for q in range(BW // 16)]

    def idx_fetch(s, u):
        return pltpu.make_async_copy(
            idxT_hbm.at[s, pl.ds(b0, BW)], idx_v.at[u], sem_i[u])

    def gathers(u):
        return [
            pltpu.make_async_copy(
                table_hbm.at[idx_v.at[u, pl.ds(h * (BW // 2), BW // 2)]],
                buf_g.at[u, pl.ds(h * (BW // 2), BW // 2)],
                sem_g[u],
            )
            for h in range(2)
        ]

    def writebacks(s, u):
        return [
            pltpu.make_async_copy(
                buf_o.at[u, pl.ds(8 * ct, 8)],
                out_hbm.at[s, ct, wid],
                sem_o[u],
            )
            for ct in range(8)
        ]

    def compute(s, u, uo):
        s_vec = jnp.zeros((16,), jnp.int32) + s

        @plsc.parallel_loop(0, 4 * 16, unroll=2)
        def _j(j):
            cvec = ((lanes + j) & 15) + (j & 48)
            pvr = plsc.load_gather(pos_v, [s_vec, cvec])
            for q in range(BW // 16):
                x = plsc.load_gather(buf_g.at[u], [bvecs[q], cvec])
                plsc.store_scatter(
                    buf_o.at[uo], [cvec, bvecs[q]], x * SCALE + pvr)

    for u in range(4):
        idx_fetch(u, u).start()
    for u in range(4):
        idx_fetch(u, u).wait()
        for gth in gathers(u):
            gth.start()

    @pl.loop(0, SEQ, step=4)
    def _s(g):
        for u in range(4):
            s = g + u
            uo = u % 2
            for gth in gathers(u):
                gth.wait()

            @pl.when(s + 4 < SEQ)
            def _prefetch_idx():
                idx_fetch(s + 4, u).start()

            @pl.when(s >= 2)
            def _wait_prev_writeback():
                for wb in writebacks(s, uo):
                    wb.wait()

            compute(s, u, uo)
            for wb in writebacks(s, uo):
                wb.start()

            @pl.when(s + 4 < SEQ)
            def _next_gather():
                idx_fetch(s + 4, u).wait()
                for gth in gathers(u):
                    gth.start()

    for u in range(2):
        for wb in writebacks(SEQ - 2 + u, u):
            wb.wait()


@jax.jit
def _embed(idxT, pos200, table):
    mesh = plsc.VectorSubcoreMesh(core_axis_name="c", subcore_axis_name="s")
    kfn = pl.kernel(
        _sc_body,
        out_type=jax.ShapeDtypeStruct((SEQ, 8, NW, 8, BW), jnp.float32),
        mesh=mesh,
        scratch_types=[
            pltpu.VMEM((4, BW), jnp.int32),
            pltpu.VMEM((4, BW, D), jnp.float32),
            pltpu.VMEM((2, D, BW), jnp.float32),
            pltpu.VMEM((SEQ, D), jnp.float32),
        ] + [pltpu.SemaphoreType.DMA] * 10,
        compiler_params=pltpu.CompilerParams(
            needs_layout_passes=False,
            use_tc_tiling_on_sc=False,
        ),
    )
    return kfn(idxT, pos200, table)


def kernel(inputs, table, pos_encoding):
    idxT = inputs.T
    pos200 = pos_encoding[:SEQ]
    out5 = _embed(idxT, pos200, table)
    return out5.transpose(2, 4, 0, 1, 3).reshape(BATCH, SEQ, D)


# speedup vs baseline: 1.0102x; 1.0060x over previous
import jax
import jax.numpy as jnp
from jax import lax
from jax.experimental import pallas as pl
from jax.experimental.pallas import tpu as pltpu
from jax.experimental.pallas import tpu_sc as plsc

VOCAB = 100000
D = 64
BATCH = 4096
SEQ = 200
NC, NS = 2, 16
NW = NC * NS
BW = BATCH // NW
SCALE = 8.0


def _sc_body(idxT_hbm, pos_hbm, table_hbm, out_hbm,
             idx_v, buf_g, buf_o, pos_v,
             sem_i0, sem_i1, sem_i2, sem_i3,
             sem_g0, sem_g1, sem_g2, sem_g3, sem_o0, sem_o1):
    wid = lax.axis_index("s") * NC + lax.axis_index("c")
    b0 = wid * BW
    sem_i = (sem_i0, sem_i1, sem_i2, sem_i3)
    sem_g = (sem_g0, sem_g1, sem_g2, sem_g3)
    sem_o = (sem_o0, sem_o1)

    pltpu.sync_copy(pos_hbm, pos_v)

    lanes = lax.iota(jnp.int32, 16)
    bvecs = [lanes + 16 * q for q in range(BW // 16)]

    def idx_fetch(s, u):
        return pltpu.make_async_copy(
            idxT_hbm.at[s, pl.ds(b0, BW)], idx_v.at[u], sem_i[u])

    def gather(u):
        return pltpu.make_async_copy(
            table_hbm.at[idx_v.at[u]], buf_g.at[u], sem_g[u])

    def writebacks(s, u):
        return [
            pltpu.make_async_copy(
                buf_o.at[u, pl.ds(8 * ct, 8)],
                out_hbm.at[s, ct, wid],
                sem_o[u],
            )
            for ct in range(8)
        ]

    def compute(s, u, uo):
        s_vec = jnp.zeros((16,), jnp.int32) + s

        @plsc.parallel_loop(0, 4 * 16, unroll=2)
        def _j(j):
            cvec = ((lanes + j) & 15) + (j & 48)
            pvr = plsc.load_gather(pos_v, [s_vec, cvec])
            for q in range(BW // 16):
                x = plsc.load_gather(buf_g.at[u], [bvecs[q], cvec])
                plsc.store_scatter(
                    buf_o.at[uo], [cvec, bvecs[q]], x * SCALE + pvr)

    for u in range(4):
        idx_fetch(u, u).start()
    for u in range(4):
        idx_fetch(u, u).wait()
        gather(u).start()

    @pl.loop(0, SEQ, step=4)
    def _s(g):
        for u in range(4):
            s = g + u
            uo = u % 2
            gather(u).wait()

            @pl.when(s + 4 < SEQ)
            def _prefetch_idx():
                idx_fetch(s + 4, u).start()

            @pl.when(s >= 2)
            def _wait_prev_writeback():
                for wb in writebacks(s, uo):
                    wb.wait()

            compute(s, u, uo)
            for wb in writebacks(s, uo):
                wb.start()

            @pl.when(s + 4 < SEQ)
            def _next_gather():
                idx_fetch(s + 4, u).wait()
                gather(u).start()

    for u in range(2):
        for wb in writebacks(SEQ - 2 + u, u):
            wb.wait()


@jax.jit
def _embed(idxT, pos200, table):
    mesh = plsc.VectorSubcoreMesh(core_axis_name="c", subcore_axis_name="s")
    kfn = pl.kernel(
        _sc_body,
        out_type=jax.ShapeDtypeStruct((SEQ, 8, NW, 8, BW), jnp.float32),
        mesh=mesh,
        scratch_types=[
            pltpu.VMEM((4, BW), jnp.int32),
            pltpu.VMEM((4, BW, D), jnp.float32),
            pltpu.VMEM((2, D, BW), jnp.float32),
            pltpu.VMEM((SEQ, D), jnp.float32),
        ] + [pltpu.SemaphoreType.DMA] * 10,
        compiler_params=pltpu.CompilerParams(
            needs_layout_passes=False,
            use_tc_tiling_on_sc=False,
        ),
    )
    return kfn(idxT, pos200, table)


def kernel(inputs, table, pos_encoding):
    idxT = inputs.T
    pos200 = pos_encoding[:SEQ]
    out5 = _embed(idxT, pos200, table)
    return out5.transpose(2, 4, 0, 1, 3).reshape(BATCH, SEQ, D)
